# baseline (speedup 1.0000x reference)
import functools

import numpy as np

import jax
import jax.numpy as jnp
from jax.experimental import pallas as pl
from jax.experimental.pallas import tpu as pltpu


def _resblock_body(mask_ref, x_ref, w1_ref, b1_ref, w2_ref, b2_ref, o_ref,
                   *, W, nb, G):
    C = x_ref.shape[1]
    HW = x_ref.shape[2]
    GHW = G * HW

    m_xm = mask_ref[0:1, :]
    m_xp = mask_ref[1:2, :]
    m_yu = mask_ref[2:3, :]
    m_yd = mask_ref[3:4, :]
    b1 = b1_ref[...]
    b2 = b2_ref[...]

    def conv(inp_bf, w_ref):
        v = pltpu.bitcast(inp_bf, jnp.int32)
        cm = pltpu.bitcast(pltpu.roll(v, shift=1, axis=1) & m_xm,
                           jnp.bfloat16)
        cp = pltpu.bitcast(pltpu.roll(v, shift=GHW - 1, axis=1) & m_xp,
                           jnp.bfloat16)
        xc = jnp.concatenate([cm, inp_bf, cp], axis=0)
        p = jnp.dot(w_ref[...], xc, preferred_element_type=jnp.float32)
        p0 = pltpu.bitcast(p[:C].astype(jnp.bfloat16), jnp.int32)
        p2 = pltpu.bitcast(p[2 * C:].astype(jnp.bfloat16), jnp.int32)
        up = pltpu.bitcast(pltpu.roll(p0, shift=W, axis=1) & m_yu,
                           jnp.bfloat16)
        dn = pltpu.bitcast(pltpu.roll(p2, shift=GHW - W, axis=1) & m_yd,
                           jnp.bfloat16)
        return p[C:2 * C].astype(jnp.bfloat16) + (up + dn)

    for i in range(nb // G):
        xb = jnp.concatenate(
            [x_ref[G * i + j].astype(jnp.bfloat16) for j in range(G)], axis=1)
        hidden = jnp.maximum(conv(xb, w1_ref) + b1, 0)
        acc = (conv(hidden, w2_ref) + b2).astype(jnp.float32)
        for j in range(G):
            o_ref[G * i + j] = x_ref[G * i + j] + acc[:, j * HW:(j + 1) * HW]


def _edge_masks(H, W, G):
    q = np.arange(H * W, dtype=np.int32)
    w = q % W
    rows = [
        w >= 1,
        w <= W - 2,
        q >= W,
        q < H * W - W,
    ]
    rows = [np.where(r, np.int32(-1), np.int32(0)) for r in rows]
    rows += [np.zeros((H * W,), np.int32)] * 4
    return jnp.asarray(np.tile(np.stack(rows, axis=0), (1, G)))


def _row_grouped(wt, C):
    return jnp.transpose(wt, (2, 0, 3, 1)).reshape(3 * C, 3 * C).astype(
        jnp.bfloat16)


def kernel(x, w1, b1, w2, b2):
    N, C, H, W = x.shape
    HW = H * W

    nb = 16
    while N % nb:
        nb //= 2
    G = 4 if nb % 4 == 0 else (2 if nb % 2 == 0 else 1)
    grid = (N // nb,)

    x_r = x.reshape(N, C, HW)
    wk1 = _row_grouped(w1, C)
    wk2 = _row_grouped(w2, C)
    b1_k = b1.reshape(C, 1).astype(jnp.bfloat16)
    b2_k = b2.reshape(C, 1).astype(jnp.bfloat16)
    masks = _edge_masks(H, W, G)

    body = functools.partial(_resblock_body, W=W, nb=nb, G=G)
    out = pl.pallas_call(
        body,
        out_shape=jax.ShapeDtypeStruct((N, C, HW), x.dtype),
        grid=grid,
        in_specs=[
            pl.BlockSpec((8, G * HW), lambda g: (0, 0)),
            pl.BlockSpec((nb, C, HW), lambda g: (g, 0, 0)),
            pl.BlockSpec((3 * C, 3 * C), lambda g: (0, 0)),
            pl.BlockSpec((C, 1), lambda g: (0, 0)),
            pl.BlockSpec((3 * C, 3 * C), lambda g: (0, 0)),
            pl.BlockSpec((C, 1), lambda g: (0, 0)),
        ],
        out_specs=pl.BlockSpec((nb, C, HW), lambda g: (g, 0, 0)),
        compiler_params=pltpu.CompilerParams(
            dimension_semantics=("parallel",),
            vmem_limit_bytes=48 << 20,
        ),
    )(masks, x_r, wk1, b1_k, wk2, b2_k)

    return out.reshape(N, C, H, W)


# speedup vs baseline: 1.0453x; 1.0453x over previous
import functools

import numpy as np

import jax
import jax.numpy as jnp
from jax.experimental import pallas as pl
from jax.experimental.pallas import tpu as pltpu


def _resblock_body(mask_ref, x_ref, w1_ref, b1_ref, w2_ref, b2_ref, o_ref,
                   *, W, nb, G):
    C = x_ref.shape[1]
    HW = x_ref.shape[2]
    GHW = G * HW

    m_xm = mask_ref[0:1, :]
    m_xp = mask_ref[1:2, :]
    m_yu = mask_ref[2:3, :]
    m_yd = mask_ref[3:4, :]
    b1 = b1_ref[...]
    b2 = b2_ref[...]

    def conv(inp_bf, w_ref):
        v = pltpu.bitcast(inp_bf, jnp.int32)
        cm = pltpu.bitcast(pltpu.roll(v, shift=1, axis=1) & m_xm,
                           jnp.bfloat16)
        cp = pltpu.bitcast(pltpu.roll(v, shift=GHW - 1, axis=1) & m_xp,
                           jnp.bfloat16)
        xc = jnp.concatenate([cm, inp_bf, cp], axis=0)
        p = jnp.dot(w_ref[...], xc, preferred_element_type=jnp.float32)
        p0 = pltpu.bitcast(p[:C].astype(jnp.bfloat16), jnp.int32)
        p2 = pltpu.bitcast(p[2 * C:].astype(jnp.bfloat16), jnp.int32)
        up = pltpu.bitcast(pltpu.roll(p0, shift=W, axis=1) & m_yu,
                           jnp.bfloat16)
        dn = pltpu.bitcast(pltpu.roll(p2, shift=GHW - W, axis=1) & m_yd,
                           jnp.bfloat16)
        return p[C:2 * C].astype(jnp.bfloat16) + (up + dn)

    for i in range(nb // G):
        xb = jnp.concatenate(
            [x_ref[G * i + j].astype(jnp.bfloat16) for j in range(G)], axis=1)
        hidden = jnp.maximum(conv(xb, w1_ref) + b1, 0)
        acc = (conv(hidden, w2_ref) + b2).astype(jnp.float32)
        for j in range(G):
            o_ref[G * i + j] = x_ref[G * i + j] + acc[:, j * HW:(j + 1) * HW]


def _edge_masks(H, W, G):
    q = np.arange(H * W, dtype=np.int32)
    w = q % W
    rows = [
        w >= 1,
        w <= W - 2,
        q >= W,
        q < H * W - W,
    ]
    rows = [np.where(r, np.int32(-1), np.int32(0)) for r in rows]
    rows += [np.zeros((H * W,), np.int32)] * 4
    return jnp.asarray(np.tile(np.stack(rows, axis=0), (1, G)))


def _row_grouped(wt, C):
    return jnp.transpose(wt, (2, 0, 3, 1)).reshape(3 * C, 3 * C).astype(
        jnp.bfloat16)


def kernel(x, w1, b1, w2, b2):
    N, C, H, W = x.shape
    HW = H * W

    nb = 8
    while N % nb:
        nb //= 2
    G = 4 if nb % 4 == 0 else (2 if nb % 2 == 0 else 1)
    grid = (N // nb,)

    x_r = x.reshape(N, C, HW)
    wk1 = _row_grouped(w1, C)
    wk2 = _row_grouped(w2, C)
    b1_k = b1.reshape(C, 1).astype(jnp.bfloat16)
    b2_k = b2.reshape(C, 1).astype(jnp.bfloat16)
    masks = _edge_masks(H, W, G)

    body = functools.partial(_resblock_body, W=W, nb=nb, G=G)
    out = pl.pallas_call(
        body,
        out_shape=jax.ShapeDtypeStruct((N, C, HW), x.dtype),
        grid=grid,
        in_specs=[
            pl.BlockSpec((8, G * HW), lambda g: (0, 0)),
            pl.BlockSpec((nb, C, HW), lambda g: (g, 0, 0)),
            pl.BlockSpec((3 * C, 3 * C), lambda g: (0, 0)),
            pl.BlockSpec((C, 1), lambda g: (0, 0)),
            pl.BlockSpec((3 * C, 3 * C), lambda g: (0, 0)),
            pl.BlockSpec((C, 1), lambda g: (0, 0)),
        ],
        out_specs=pl.BlockSpec((nb, C, HW), lambda g: (g, 0, 0)),
        compiler_params=pltpu.CompilerParams(
            dimension_semantics=("parallel",),
            vmem_limit_bytes=48 << 20,
        ),
    )(masks, x_r, wk1, b1_k, wk2, b2_k)

    return out.reshape(N, C, H, W)
